# Initial kernel scaffold; baseline (speedup 1.0000x reference)
import jax
import jax.numpy as jnp
from jax.experimental import pallas as pl


def kernel(x, pos, edge_index, params):
    raise NotImplementedError("write your pallas kernel here")


# speedup vs baseline: 4.8211x; 4.8211x over previous
import functools

import jax
import jax.numpy as jnp
from jax import lax
from jax.experimental import pallas as pl
from jax.experimental.pallas import tpu as pltpu
from jax.experimental.pallas import tpu_sc as plsc

N = 10000
E = 160000
D = 256
HH = 128
LAYERS = 6
NT = 5
EPS = 1e-5

NSC = 2
NTEC = 16
CH = 128
N_PAD = 10240
RPT = N_PAD // NTEC
EP = -(-E // (NSC * NTEC * CH)) * (NSC * NTEC * CH)
NCHUNK = EP // (NTEC * CH)
NCHUNK2 = EP // (NSC * NTEC * CH)

_mesh = plsc.VectorSubcoreMesh(core_axis_name="c", subcore_axis_name="s")


def _zero_zb(zb, width):
    for i in range(16):
        for k in range(width // 16):
            zb[i, pl.ds(16 * k, 16)] = jnp.zeros((16,), jnp.float32)


@functools.partial(
    pl.kernel,
    out_type=jax.ShapeDtypeStruct((NSC * NTEC, RPT, HH), jnp.float32),
    mesh=_mesh,
    scratch_types=[
        pltpu.VMEM((NCHUNK2, CH), jnp.int32),
        pltpu.VMEM((CH, HH), jnp.float32),
        pltpu.VMEM((16, HH), jnp.float32),
        pltpu.VMEM_SHARED((N_PAD, HH), jnp.float32),
    ],
)
def _deg_kernel(dst_hbm, ones_hbm, out_hbm, dst_v, ones_v, zb_v, acc):
    c = lax.axis_index("c")
    s = lax.axis_index("s")
    w = c * NTEC + s
    _zero_zb(zb_v, HH)
    pltpu.sync_copy(ones_hbm, ones_v)

    def zrow(i, carry):
        pltpu.sync_copy(zb_v, acc.at[pl.ds(s * RPT + i * 16, 16)])
        return carry
    lax.fori_loop(0, RPT // 16, zrow, 0)
    pltpu.sync_copy(dst_hbm.at[w], dst_v)
    plsc.subcore_barrier()

    def body(j, carry):
        pltpu.sync_copy(ones_v, acc.at[dst_v.at[j]], add=True)
        return carry
    lax.fori_loop(0, NCHUNK2, body, 0)
    plsc.subcore_barrier()
    pltpu.sync_copy(acc.at[pl.ds(s * RPT, RPT)], out_hbm.at[w])


@functools.partial(
    pl.kernel,
    out_type=jax.ShapeDtypeStruct((NSC * NTEC, RPT, HH), jnp.float32),
    mesh=_mesh,
    scratch_types=[
        pltpu.VMEM((NCHUNK, CH), jnp.int32),
        pltpu.VMEM((NCHUNK, CH), jnp.int32),
        pltpu.VMEM((CH, HH), jnp.float32),
        pltpu.VMEM((16, HH), jnp.float32),
        pltpu.VMEM_SHARED((N_PAD, HH), jnp.float32),
        pltpu.SemaphoreType.DMA,
    ],
)
def _agg_kernel(xtl_hbm, xtr_hbm, src_hbm, dst_hbm, out_hbm,
                src_v, dst_v, rows_v, zb_v, acc, sem):
    c = lax.axis_index("c")
    s = lax.axis_index("s")
    w = c * NTEC + s
    _zero_zb(zb_v, HH)
    pltpu.sync_copy(src_hbm.at[s], src_v)
    pltpu.sync_copy(dst_hbm.at[s], dst_v)

    def run(xth):
        def init(i, carry):
            row0 = s * RPT + i * 16

            @pl.when(row0 < N)
            def _():
                pltpu.sync_copy(xth.at[pl.ds(row0, 16)],
                                acc.at[pl.ds(row0, 16)])

            @pl.when(row0 >= N)
            def _():
                pltpu.sync_copy(zb_v, acc.at[pl.ds(row0, 16)])
            return carry
        lax.fori_loop(0, RPT // 16, init, 0)
        plsc.subcore_barrier()

        def body(j, carry):
            pltpu.async_copy(xth.at[src_v.at[j]], rows_v, sem).wait()
            pltpu.sync_copy(rows_v, acc.at[dst_v.at[j]], add=True)
            return carry
        lax.fori_loop(0, NCHUNK, body, 0)

    @pl.when(c == 0)
    def _():
        run(xtl_hbm)

    @pl.when(c == 1)
    def _():
        run(xtr_hbm)

    plsc.subcore_barrier()
    pltpu.sync_copy(acc.at[pl.ds(s * RPT, RPT)], out_hbm.at[w])


def _bn(t, g, b):
    m = jnp.mean(t, axis=0, keepdims=True)
    v = jnp.mean(t * t, axis=0, keepdims=True) - m * m
    return (t - m) * lax.rsqrt(v + EPS) * g + b


def _silu(t):
    return t * lax.logistic(t)


def _prologue_body(x_ref, we_ref, wb_ref, g_ref, b_ref, cnt_ref, w0_ref,
                   dis_ref, h_ref, xtl_ref, xtr_ref):
    dis = lax.rsqrt(cnt_ref[...] + 1.0)
    t = jnp.dot(x_ref[...], we_ref[...],
                preferred_element_type=jnp.float32) + wb_ref[...]
    h = _silu(_bn(t, g_ref[...], b_ref[...]))
    xt = jnp.dot(h, w0_ref[...], preferred_element_type=jnp.float32) * dis
    dis_ref[...] = dis
    h_ref[...] = h
    xtl_ref[...] = xt[:, :HH]
    xtr_ref[...] = xt[:, HH:]


_TC_PARAMS = pltpu.CompilerParams(vmem_limit_bytes=66_000_000)

_prologue_call = pl.pallas_call(
    _prologue_body,
    compiler_params=_TC_PARAMS,
    out_shape=[
        jax.ShapeDtypeStruct((N, 1), jnp.float32),
        jax.ShapeDtypeStruct((N, D), jnp.float32),
        jax.ShapeDtypeStruct((N, HH), jnp.float32),
        jax.ShapeDtypeStruct((N, HH), jnp.float32),
    ],
)


def _combine_body(with_res, with_next,
                  aggl_ref, aggr_ref, h_ref, dis_ref,
                  b_ref, g_ref, be_ref, *rest):
    if with_next:
        wn_ref, hn_ref, xtnl_ref, xtnr_ref = rest
    else:
        (hn_ref,) = rest
    dis = dis_ref[...]
    t = jnp.concatenate([aggl_ref[:N, :], aggr_ref[:N, :]],
                        axis=1) * dis + b_ref[...]
    hn = _silu(_bn(t, g_ref[...], be_ref[...]))
    if with_res:
        hn = hn + h_ref[...]
    hn_ref[...] = hn
    if with_next:
        xtn = jnp.dot(hn, wn_ref[...], preferred_element_type=jnp.float32) * dis
        xtnl_ref[...] = xtn[:, :HH]
        xtnr_ref[...] = xtn[:, HH:]


def _make_combine(with_res, with_next):
    outs = [jax.ShapeDtypeStruct((N, D), jnp.float32)]
    if with_next:
        outs += [jax.ShapeDtypeStruct((N, HH), jnp.float32),
                 jax.ShapeDtypeStruct((N, HH), jnp.float32)]
    return pl.pallas_call(
        functools.partial(_combine_body, with_res, with_next),
        compiler_params=_TC_PARAMS,
        out_shape=outs,
    )


_combine_first = _make_combine(False, True)
_combine_mid = _make_combine(True, True)
_combine_last = _make_combine(True, False)


def _head_body(h_ref, w1_ref, b1_ref, g1_ref, be1_ref,
               w2_ref, b2_ref, g2_ref, be2_ref, w3_ref, b3_ref, o_ref):
    t = jnp.dot(h_ref[...], w1_ref[...],
                preferred_element_type=jnp.float32) + b1_ref[...]
    t = _silu(_bn(t, g1_ref[...], be1_ref[...]))
    t = jnp.dot(t, w2_ref[...], preferred_element_type=jnp.float32) + b2_ref[...]
    t = _silu(_bn(t, g2_ref[...], be2_ref[...]))
    o_ref[...] = jnp.dot(t, w3_ref[...],
                         preferred_element_type=jnp.float32) + b3_ref[...]


_head_call = pl.pallas_call(
    _head_body,
    compiler_params=_TC_PARAMS,
    out_shape=jax.ShapeDtypeStruct((N, HH), jnp.float32),
)


def _row(a):
    return a.reshape(1, -1)


def kernel(x, pos, edge_index, params):
    p = params
    src = edge_index[0]
    dst = edge_index[1]
    pad = EP - E
    srcp = jnp.concatenate([src, jnp.zeros((pad,), jnp.int32)])
    dstp = jnp.concatenate([dst, jnp.full((pad,), N_PAD - 1, jnp.int32)])
    src_slab = srcp.reshape(NTEC, NCHUNK, CH)
    dst_slab = dstp.reshape(NTEC, NCHUNK, CH)
    dst_slab2 = dstp.reshape(NSC * NTEC, NCHUNK2, CH)

    ones_rows = jnp.ones((CH, HH), jnp.float32)
    deg_out = _deg_kernel(dst_slab2, ones_rows).reshape(NSC, N_PAD, HH)
    cnt = deg_out[0, :N, 0:1] + deg_out[1, :N, 0:1]
    dis, h, xtl, xtr = _prologue_call(
        x, p["embed_W"], _row(p["embed_b"]), _row(p["bn_embed_g"]),
        _row(p["bn_embed_b"]), cnt, p["gcn_W"][0])

    for i in range(LAYERS):
        agg = _agg_kernel(xtl, xtr, src_slab, dst_slab)
        agg = agg.reshape(NSC, N_PAD, HH)
        args = (agg[0], agg[1], h, dis,
                _row(p["gcn_b"][i]), _row(p["bn_g"][i]), _row(p["bn_b"][i]))
        if i == 0:
            h, xtl, xtr = _combine_first(*args, p["gcn_W"][i + 1])
        elif i < LAYERS - 1:
            h, xtl, xtr = _combine_mid(*args, p["gcn_W"][i + 1])
        else:
            (h,) = _combine_last(*args)

    def head(hp, dout):
        w3 = jnp.pad(hp["W3"], ((0, 0), (0, HH - dout)))
        b3 = jnp.pad(hp["b3"], ((0, HH - dout)))
        o = _head_call(h, hp["W1"], _row(hp["b1"]), _row(hp["g1"]),
                       _row(hp["be1"]), hp["W2"], _row(hp["b2"]),
                       _row(hp["g2"]), _row(hp["be2"]), w3, _row(b3))
        return o[:, :dout]

    atom = head(p["head_atom"], NT)
    posoff = head(p["head_pos"], 3)
    return (atom, posoff)


# speedup vs baseline: 5.2463x; 1.0882x over previous
import functools

import jax
import jax.numpy as jnp
from jax import lax
from jax.experimental import pallas as pl
from jax.experimental.pallas import tpu as pltpu
from jax.experimental.pallas import tpu_sc as plsc

N = 10000
E = 160000
D = 256
HH = 128
LAYERS = 6
NT = 5
EPS = 1e-5

NSC = 2
NTEC = 16
CH = 128
N_PAD = 10240
RPT = N_PAD // NTEC
EP = -(-E // (NSC * NTEC * CH)) * (NSC * NTEC * CH)
NCHUNK = EP // (NTEC * CH)
NCHUNK2 = EP // (NSC * NTEC * CH)
NBUF = 2
NPASS = 2
HP = NCHUNK // NPASS

_mesh = plsc.VectorSubcoreMesh(core_axis_name="c", subcore_axis_name="s")


def _zero_zb(zb, width):
    for i in range(16):
        for k in range(width // 16):
            zb[i, pl.ds(16 * k, 16)] = jnp.zeros((16,), jnp.float32)


@functools.partial(
    pl.kernel,
    out_type=jax.ShapeDtypeStruct((NSC * NTEC, RPT, HH), jnp.float32),
    mesh=_mesh,
    scratch_types=[
        pltpu.VMEM((NCHUNK2, CH), jnp.int32),
        pltpu.VMEM((CH, HH), jnp.float32),
        pltpu.VMEM((16, HH), jnp.float32),
        pltpu.VMEM_SHARED((N_PAD, HH), jnp.float32),
    ],
)
def _deg_kernel(dst_hbm, ones_hbm, out_hbm, dst_v, ones_v, zb_v, acc):
    c = lax.axis_index("c")
    s = lax.axis_index("s")
    w = c * NTEC + s
    _zero_zb(zb_v, HH)
    pltpu.sync_copy(ones_hbm, ones_v)

    def zrow(i, carry):
        pltpu.sync_copy(zb_v, acc.at[pl.ds(s * RPT + i * 16, 16)])
        return carry
    lax.fori_loop(0, RPT // 16, zrow, 0)
    pltpu.sync_copy(dst_hbm.at[w], dst_v)
    plsc.subcore_barrier()

    def body(j, carry):
        pltpu.sync_copy(ones_v, acc.at[dst_v.at[j]], add=True)
        return carry
    lax.fori_loop(0, NCHUNK2, body, 0)
    plsc.subcore_barrier()
    pltpu.sync_copy(acc.at[pl.ds(s * RPT, RPT)], out_hbm.at[w])


@functools.partial(
    pl.kernel,
    out_type=jax.ShapeDtypeStruct((NSC * NTEC, RPT, HH), jnp.float32),
    mesh=_mesh,
    scratch_types=[
        pltpu.VMEM((HP, CH), jnp.int32),
        pltpu.VMEM((HP, CH), jnp.int32),
        pltpu.VMEM((NBUF, CH, HH), jnp.float32),
        pltpu.VMEM_SHARED((N_PAD, HH), jnp.float32),
        pltpu.SemaphoreType.DMA,
    ],
)
def _agg_kernel(xtl_hbm, xtr_hbm, src_hbm, dst_hbm, out_hbm,
                src_v, dst_v, rows_v, acc, sem):
    c = lax.axis_index("c")
    s = lax.axis_index("s")
    w = c * NTEC + s

    def run(xth):
        def init(i, carry):
            row0 = s * RPT + i * 16

            @pl.when(row0 < N)
            def _():
                pltpu.sync_copy(xth.at[pl.ds(row0, 16)],
                                acc.at[pl.ds(row0, 16)])
            return carry
        lax.fori_loop(0, RPT // 16, init, 0)
        plsc.subcore_barrier()

        for p in range(NPASS):
            pltpu.sync_copy(src_hbm.at[s * NPASS + p], src_v)
            pltpu.sync_copy(dst_hbm.at[s * NPASS + p], dst_v)
            for b in range(NBUF):
                pltpu.make_async_copy(xth.at[src_v.at[b]], rows_v.at[b],
                                      sem).start()

            def outer(i, carry):
                for b in range(NBUF):
                    pltpu.make_async_copy(xth.at[src_v.at[b]],
                                          rows_v.at[b], sem).wait()
                for b in range(NBUF):
                    j = i * NBUF + b
                    pltpu.sync_copy(rows_v.at[b], acc.at[dst_v.at[j]],
                                    add=True)

                    @pl.when(j + NBUF < HP)
                    def _():
                        pltpu.make_async_copy(xth.at[src_v.at[j + NBUF]],
                                              rows_v.at[b], sem).start()
                return carry
            lax.fori_loop(0, HP // NBUF, outer, 0)

    @pl.when(c == 0)
    def _():
        run(xtl_hbm)

    @pl.when(c == 1)
    def _():
        run(xtr_hbm)

    plsc.subcore_barrier()
    pltpu.sync_copy(acc.at[pl.ds(s * RPT, RPT)], out_hbm.at[w])


def _bn(t, g, b):
    m = jnp.mean(t, axis=0, keepdims=True)
    v = jnp.mean(t * t, axis=0, keepdims=True) - m * m
    return (t - m) * lax.rsqrt(v + EPS) * g + b


def _silu(t):
    return t * lax.logistic(t)


def _prologue_body(x_ref, we_ref, wb_ref, g_ref, b_ref, cnt_ref, w0_ref,
                   dis_ref, h_ref, xtl_ref, xtr_ref):
    dis = lax.rsqrt(cnt_ref[...] + 1.0)
    t = jnp.dot(x_ref[...], we_ref[...],
                preferred_element_type=jnp.float32) + wb_ref[...]
    h = _silu(_bn(t, g_ref[...], b_ref[...]))
    xt = jnp.dot(h, w0_ref[...], preferred_element_type=jnp.float32) * dis
    dis_ref[...] = dis
    h_ref[...] = h
    xtl_ref[...] = xt[:, :HH]
    xtr_ref[...] = xt[:, HH:]


_TC_PARAMS = pltpu.CompilerParams(vmem_limit_bytes=66_000_000)

_prologue_call = pl.pallas_call(
    _prologue_body,
    compiler_params=_TC_PARAMS,
    out_shape=[
        jax.ShapeDtypeStruct((N, 1), jnp.float32),
        jax.ShapeDtypeStruct((N, D), jnp.float32),
        jax.ShapeDtypeStruct((N, HH), jnp.float32),
        jax.ShapeDtypeStruct((N, HH), jnp.float32),
    ],
)


def _combine_body(with_res, with_next,
                  aggl_ref, aggr_ref, h_ref, dis_ref,
                  b_ref, g_ref, be_ref, *rest):
    if with_next:
        wn_ref, hn_ref, xtnl_ref, xtnr_ref = rest
    else:
        (hn_ref,) = rest
    dis = dis_ref[...]
    t = jnp.concatenate([aggl_ref[:N, :], aggr_ref[:N, :]],
                        axis=1) * dis + b_ref[...]
    hn = _silu(_bn(t, g_ref[...], be_ref[...]))
    if with_res:
        hn = hn + h_ref[...]
    hn_ref[...] = hn
    if with_next:
        xtn = jnp.dot(hn, wn_ref[...], preferred_element_type=jnp.float32) * dis
        xtnl_ref[...] = xtn[:, :HH]
        xtnr_ref[...] = xtn[:, HH:]


def _make_combine(with_res, with_next):
    outs = [jax.ShapeDtypeStruct((N, D), jnp.float32)]
    if with_next:
        outs += [jax.ShapeDtypeStruct((N, HH), jnp.float32),
                 jax.ShapeDtypeStruct((N, HH), jnp.float32)]
    return pl.pallas_call(
        functools.partial(_combine_body, with_res, with_next),
        compiler_params=_TC_PARAMS,
        out_shape=outs,
    )


_combine_first = _make_combine(False, True)
_combine_mid = _make_combine(True, True)
_combine_last = _make_combine(True, False)


def _head_body(h_ref, w1_ref, b1_ref, g1_ref, be1_ref,
               w2_ref, b2_ref, g2_ref, be2_ref, w3_ref, b3_ref, o_ref):
    t = jnp.dot(h_ref[...], w1_ref[...],
                preferred_element_type=jnp.float32) + b1_ref[...]
    t = _silu(_bn(t, g1_ref[...], be1_ref[...]))
    t = jnp.dot(t, w2_ref[...], preferred_element_type=jnp.float32) + b2_ref[...]
    t = _silu(_bn(t, g2_ref[...], be2_ref[...]))
    o_ref[...] = jnp.dot(t, w3_ref[...],
                         preferred_element_type=jnp.float32) + b3_ref[...]


_head_call = pl.pallas_call(
    _head_body,
    compiler_params=_TC_PARAMS,
    out_shape=jax.ShapeDtypeStruct((N, HH), jnp.float32),
)


def _row(a):
    return a.reshape(1, -1)


def kernel(x, pos, edge_index, params):
    p = params
    src = edge_index[0]
    dst = edge_index[1]
    pad = EP - E
    srcp = jnp.concatenate([src, jnp.zeros((pad,), jnp.int32)])
    dstp = jnp.concatenate([dst, jnp.full((pad,), N_PAD - 1, jnp.int32)])
    src_slab = srcp.reshape(NTEC * NPASS, HP, CH)
    dst_slab = dstp.reshape(NTEC * NPASS, HP, CH)
    dst_slab2 = dstp.reshape(NSC * NTEC, NCHUNK2, CH)

    ones_rows = jnp.ones((CH, HH), jnp.float32)
    deg_out = _deg_kernel(dst_slab2, ones_rows).reshape(NSC, N_PAD, HH)
    cnt = deg_out[0, :N, 0:1] + deg_out[1, :N, 0:1]
    dis, h, xtl, xtr = _prologue_call(
        x, p["embed_W"], _row(p["embed_b"]), _row(p["bn_embed_g"]),
        _row(p["bn_embed_b"]), cnt, p["gcn_W"][0])

    for i in range(LAYERS):
        agg = _agg_kernel(xtl, xtr, src_slab, dst_slab)
        agg = agg.reshape(NSC, N_PAD, HH)
        args = (agg[0], agg[1], h, dis,
                _row(p["gcn_b"][i]), _row(p["bn_g"][i]), _row(p["bn_b"][i]))
        if i == 0:
            h, xtl, xtr = _combine_first(*args, p["gcn_W"][i + 1])
        elif i < LAYERS - 1:
            h, xtl, xtr = _combine_mid(*args, p["gcn_W"][i + 1])
        else:
            (h,) = _combine_last(*args)

    def head(hp, dout):
        w3 = jnp.pad(hp["W3"], ((0, 0), (0, HH - dout)))
        b3 = jnp.pad(hp["b3"], ((0, HH - dout)))
        o = _head_call(h, hp["W1"], _row(hp["b1"]), _row(hp["g1"]),
                       _row(hp["be1"]), hp["W2"], _row(hp["b2"]),
                       _row(hp["g2"]), _row(hp["be2"]), w3, _row(b3))
        return o[:, :dout]

    atom = head(p["head_atom"], NT)
    posoff = head(p["head_pos"], 3)
    return (atom, posoff)


# speedup vs baseline: 5.3793x; 1.0254x over previous
import functools

import jax
import jax.numpy as jnp
from jax import lax
from jax.experimental import pallas as pl
from jax.experimental.pallas import tpu as pltpu
from jax.experimental.pallas import tpu_sc as plsc

N = 10000
E = 160000
D = 256
HH = 128
LAYERS = 6
NT = 5
EPS = 1e-5

NSC = 2
NTEC = 16
CH = 128
N_PAD = 10240
RPT = N_PAD // NTEC
EP = -(-E // (NSC * NTEC * CH)) * (NSC * NTEC * CH)
NCHUNK = EP // (NTEC * CH)
NCHUNK2 = EP // (NSC * NTEC * CH)
NBUF = 2
NPASS = 2
HP = NCHUNK // NPASS
WAVE = 8

_mesh = plsc.VectorSubcoreMesh(core_axis_name="c", subcore_axis_name="s")


def _zero_zb(zb, width):
    for i in range(16):
        for k in range(width // 16):
            zb[i, pl.ds(16 * k, 16)] = jnp.zeros((16,), jnp.float32)


@functools.partial(
    pl.kernel,
    out_type=jax.ShapeDtypeStruct((NSC * NTEC, RPT, HH), jnp.float32),
    mesh=_mesh,
    scratch_types=[
        pltpu.VMEM((NCHUNK2, CH), jnp.int32),
        pltpu.VMEM((CH, HH), jnp.float32),
        pltpu.VMEM((16, HH), jnp.float32),
        pltpu.VMEM_SHARED((N_PAD, HH), jnp.float32),
        pltpu.SemaphoreType.DMA,
    ],
)
def _deg_kernel(dst_hbm, ones_hbm, out_hbm, dst_v, ones_v, zb_v, acc, sem):
    c = lax.axis_index("c")
    s = lax.axis_index("s")
    w = c * NTEC + s
    _zero_zb(zb_v, HH)
    pltpu.sync_copy(ones_hbm, ones_v)

    def zrow(i, carry):
        pltpu.make_async_copy(zb_v, acc.at[pl.ds(s * RPT + i * 16, 16)],
                              sem).start()
        return carry
    lax.fori_loop(0, RPT // 16, zrow, 0)

    def zdrain(i, carry):
        pltpu.make_async_copy(zb_v, acc.at[pl.ds(s * RPT + i * 16, 16)],
                              sem).wait()
        return carry
    lax.fori_loop(0, RPT // 16, zdrain, 0)
    pltpu.sync_copy(dst_hbm.at[w], dst_v)
    plsc.subcore_barrier()

    for wv in range(NCHUNK2 // WAVE):
        def fire(j, carry):
            pltpu.make_async_copy(
                ones_v, acc.at[dst_v.at[wv * WAVE + j]], sem).start(add=True)
            return carry
        lax.fori_loop(0, WAVE, fire, 0)

        def drain(j, carry):
            pltpu.make_async_copy(
                ones_v, acc.at[dst_v.at[wv * WAVE + j]], sem).wait()
            return carry
        lax.fori_loop(0, WAVE, drain, 0)
    plsc.subcore_barrier()
    pltpu.sync_copy(acc.at[pl.ds(s * RPT, RPT)], out_hbm.at[w])


@functools.partial(
    pl.kernel,
    out_type=jax.ShapeDtypeStruct((NSC * NTEC, RPT, HH), jnp.float32),
    mesh=_mesh,
    scratch_types=[
        pltpu.VMEM((HP, CH), jnp.int32),
        pltpu.VMEM((HP, CH), jnp.int32),
        pltpu.VMEM((NBUF, CH, HH), jnp.float32),
        pltpu.VMEM_SHARED((N_PAD, HH), jnp.float32),
        pltpu.SemaphoreType.DMA,
        pltpu.SemaphoreType.DMA,
    ],
)
def _agg_kernel(xtl_hbm, xtr_hbm, src_hbm, dst_hbm, out_hbm,
                src_v, dst_v, rows_v, acc, gsem, ssem):
    c = lax.axis_index("c")
    s = lax.axis_index("s")
    w = c * NTEC + s

    def run(xth):
        def init(i, carry):
            row0 = s * RPT + i * 16

            @pl.when(row0 < N)
            def _():
                pltpu.make_async_copy(xth.at[pl.ds(row0, 16)],
                                      acc.at[pl.ds(row0, 16)], gsem).start()
            return carry
        lax.fori_loop(0, RPT // 16, init, 0)

        def initd(i, carry):
            row0 = s * RPT + i * 16

            @pl.when(row0 < N)
            def _():
                pltpu.make_async_copy(xth.at[pl.ds(row0, 16)],
                                      acc.at[pl.ds(row0, 16)], gsem).wait()
            return carry
        lax.fori_loop(0, RPT // 16, initd, 0)
        plsc.subcore_barrier()

        for p in range(NPASS):
            pltpu.sync_copy(src_hbm.at[s * NPASS + p], src_v)
            pltpu.sync_copy(dst_hbm.at[s * NPASS + p], dst_v)
            for b in range(NBUF):
                pltpu.make_async_copy(xth.at[src_v.at[b]], rows_v.at[b],
                                      gsem).start()

            def outer(i, carry):
                for b in range(NBUF):
                    pltpu.make_async_copy(xth.at[src_v.at[b]],
                                          rows_v.at[b], gsem).wait()
                for b in range(NBUF):
                    j = i * NBUF + b
                    pltpu.make_async_copy(rows_v.at[b], acc.at[dst_v.at[j]],
                                          ssem).start(add=True)
                for b in range(NBUF):
                    j = i * NBUF + b
                    pltpu.make_async_copy(rows_v.at[b], acc.at[dst_v.at[j]],
                                          ssem).wait()

                    @pl.when(j + NBUF < HP)
                    def _():
                        pltpu.make_async_copy(xth.at[src_v.at[j + NBUF]],
                                              rows_v.at[b], gsem).start()
                return carry
            lax.fori_loop(0, HP // NBUF, outer, 0)

    @pl.when(c == 0)
    def _():
        run(xtl_hbm)

    @pl.when(c == 1)
    def _():
        run(xtr_hbm)

    plsc.subcore_barrier()
    pltpu.sync_copy(acc.at[pl.ds(s * RPT, RPT)], out_hbm.at[w])


def _bn(t, g, b):
    m = jnp.mean(t, axis=0, keepdims=True)
    v = jnp.mean(t * t, axis=0, keepdims=True) - m * m
    return (t - m) * lax.rsqrt(v + EPS) * g + b


def _silu(t):
    return t * lax.logistic(t)


def _embed_body(x_ref, we_ref, wb_ref, g_ref, b_ref, w0_ref,
                h_ref, xt0_ref):
    t = jnp.dot(x_ref[...], we_ref[...],
                preferred_element_type=jnp.float32) + wb_ref[...]
    h = _silu(_bn(t, g_ref[...], b_ref[...]))
    h_ref[...] = h
    xt0_ref[...] = jnp.dot(h, w0_ref[...], preferred_element_type=jnp.float32)


_TC_PARAMS = pltpu.CompilerParams(vmem_limit_bytes=66_000_000)

_embed_call = pl.pallas_call(
    _embed_body,
    compiler_params=_TC_PARAMS,
    out_shape=[
        jax.ShapeDtypeStruct((N, D), jnp.float32),
        jax.ShapeDtypeStruct((N, D), jnp.float32),
    ],
)


def _scale_body(cnt_ref, xt0_ref, dis_ref, xtl_ref, xtr_ref):
    dis = lax.rsqrt(cnt_ref[...] + 1.0)
    xt = xt0_ref[...] * dis
    dis_ref[...] = dis
    xtl_ref[...] = xt[:, :HH]
    xtr_ref[...] = xt[:, HH:]


_scale_call = pl.pallas_call(
    _scale_body,
    compiler_params=_TC_PARAMS,
    out_shape=[
        jax.ShapeDtypeStruct((N, 1), jnp.float32),
        jax.ShapeDtypeStruct((N, HH), jnp.float32),
        jax.ShapeDtypeStruct((N, HH), jnp.float32),
    ],
)


def _combine_body(with_res, with_next,
                  aggl_ref, aggr_ref, h_ref, dis_ref,
                  b_ref, g_ref, be_ref, *rest):
    if with_next:
        wn_ref, hn_ref, xtnl_ref, xtnr_ref = rest
    else:
        (hn_ref,) = rest
    dis = dis_ref[...]
    t = jnp.concatenate([aggl_ref[:N, :], aggr_ref[:N, :]],
                        axis=1) * dis + b_ref[...]
    hn = _silu(_bn(t, g_ref[...], be_ref[...]))
    if with_res:
        hn = hn + h_ref[...]
    hn_ref[...] = hn
    if with_next:
        xtn = jnp.dot(hn, wn_ref[...], preferred_element_type=jnp.float32) * dis
        xtnl_ref[...] = xtn[:, :HH]
        xtnr_ref[...] = xtn[:, HH:]


def _make_combine(with_res, with_next):
    outs = [jax.ShapeDtypeStruct((N, D), jnp.float32)]
    if with_next:
        outs += [jax.ShapeDtypeStruct((N, HH), jnp.float32),
                 jax.ShapeDtypeStruct((N, HH), jnp.float32)]
    return pl.pallas_call(
        functools.partial(_combine_body, with_res, with_next),
        compiler_params=_TC_PARAMS,
        out_shape=outs,
    )


_combine_first = _make_combine(False, True)
_combine_mid = _make_combine(True, True)
_combine_last = _make_combine(True, False)


def _head_body(h_ref, w1_ref, b1_ref, g1_ref, be1_ref,
               w2_ref, b2_ref, g2_ref, be2_ref, w3_ref, b3_ref, o_ref):
    t = jnp.dot(h_ref[...], w1_ref[...],
                preferred_element_type=jnp.float32) + b1_ref[...]
    t = _silu(_bn(t, g1_ref[...], be1_ref[...]))
    t = jnp.dot(t, w2_ref[...], preferred_element_type=jnp.float32) + b2_ref[...]
    t = _silu(_bn(t, g2_ref[...], be2_ref[...]))
    o_ref[...] = jnp.dot(t, w3_ref[...],
                         preferred_element_type=jnp.float32) + b3_ref[...]


_head_call = pl.pallas_call(
    _head_body,
    compiler_params=_TC_PARAMS,
    out_shape=jax.ShapeDtypeStruct((N, HH), jnp.float32),
)


def _row(a):
    return a.reshape(1, -1)


def kernel(x, pos, edge_index, params):
    p = params
    src = edge_index[0]
    dst = edge_index[1]
    pad = EP - E
    srcp = jnp.concatenate([src, jnp.zeros((pad,), jnp.int32)])
    dstp = jnp.concatenate([dst, jnp.full((pad,), N_PAD - 1, jnp.int32)])
    src_slab = srcp.reshape(NTEC * NPASS, HP, CH)
    dst_slab = dstp.reshape(NTEC * NPASS, HP, CH)
    dst_slab2 = dstp.reshape(NSC * NTEC, NCHUNK2, CH)

    ones_rows = jnp.ones((CH, HH), jnp.float32)
    deg_out = _deg_kernel(dst_slab2, ones_rows).reshape(NSC, N_PAD, HH)
    h, xt0 = _embed_call(
        x, p["embed_W"], _row(p["embed_b"]), _row(p["bn_embed_g"]),
        _row(p["bn_embed_b"]), p["gcn_W"][0])
    cnt = deg_out[0, :N, 0:1] + deg_out[1, :N, 0:1]
    dis, xtl, xtr = _scale_call(cnt, xt0)

    for i in range(LAYERS):
        agg = _agg_kernel(xtl, xtr, src_slab, dst_slab)
        agg = agg.reshape(NSC, N_PAD, HH)
        args = (agg[0], agg[1], h, dis,
                _row(p["gcn_b"][i]), _row(p["bn_g"][i]), _row(p["bn_b"][i]))
        if i == 0:
            h, xtl, xtr = _combine_first(*args, p["gcn_W"][i + 1])
        elif i < LAYERS - 1:
            h, xtl, xtr = _combine_mid(*args, p["gcn_W"][i + 1])
        else:
            (h,) = _combine_last(*args)

    def head(hp, dout):
        w3 = jnp.pad(hp["W3"], ((0, 0), (0, HH - dout)))
        b3 = jnp.pad(hp["b3"], ((0, HH - dout)))
        o = _head_call(h, hp["W1"], _row(hp["b1"]), _row(hp["g1"]),
                       _row(hp["be1"]), hp["W2"], _row(hp["b2"]),
                       _row(hp["g2"]), _row(hp["be2"]), w3, _row(b3))
        return o[:, :dout]

    atom = head(p["head_atom"], NT)
    posoff = head(p["head_pos"], 3)
    return (atom, posoff)
